# baseline (speedup 1.0000x reference)
import jax
import jax.numpy as jnp
from jax import lax
from jax.experimental import pallas as pl
from jax.experimental.pallas import tpu as pltpu

_S = 8
_V = 32
_N_OUT = 21
_CBLK = 16384


def _encoder_kernel(tok_ref, len_ref, w_ref, out_ref):
    f32, bf = jnp.float32, jnp.bfloat16
    cols = tok_ref.shape[1]

    lens = len_ref[0:1, :]
    tap_id = lax.broadcasted_iota(jnp.int32, (_S, 1), 0).astype(f32)
    toks = jnp.where(tap_id < lens, tok_ref[...].astype(bf),
                     jnp.array(-1.0, bf))

    vocab_id = lax.broadcasted_iota(jnp.int32, (_V, cols), 0).astype(bf)
    acc = jnp.zeros((_V, cols), bf)
    one, zero = jnp.array(1.0, bf), jnp.array(0.0, bf)
    for t in range(_S):
        hit = toks[t:t + 1, :] == vocab_id
        acc = acc + jnp.where(hit, one, zero)
    inv = (1.0 / jnp.maximum(lens, 1.0)).astype(bf)
    combined = acc * inv

    ones_row = jnp.full((1, cols), 1.0, bf)

    def aug(x):
        return jnp.concatenate([x, ones_row], axis=0)

    wenc_t = w_ref[0:_V, 0:_V + 1]
    w1_t = w_ref[64:128, 0:_V + 1]
    w2_t = w_ref[128:192, 0:65]
    w3_t = w_ref[192:192 + _N_OUT, 0:65]

    zero_bf = jnp.array(0.0, bf)
    sent = jnp.tanh(
        jnp.dot(wenc_t, aug(combined), preferred_element_type=f32))
    h1 = jnp.maximum(
        jnp.dot(w1_t, aug(sent.astype(bf)),
                preferred_element_type=f32).astype(bf), zero_bf)
    h2 = jnp.maximum(
        jnp.dot(w2_t, aug(h1), preferred_element_type=f32).astype(bf), zero_bf)
    out_ref[...] = jnp.dot(w3_t, aug(h2), preferred_element_type=f32)


@jax.jit
def kernel(tokens, lengths, table, wenc, benc, w1, b1, w2, b2, w3, b3):
    b, s = tokens.shape
    assert s == _S
    bf = jnp.bfloat16
    cblk = max(128, min(_CBLK, ((b + 127) // 128) * 128))
    b_pad = ((b + cblk - 1) // cblk) * cblk

    tok_t = tokens.astype(jnp.int32).T
    len_t = lengths.astype(jnp.float32).reshape(1, b)
    if b_pad != b:
        tok_t = jnp.pad(tok_t, ((0, 0), (0, b_pad - b)))
        len_t = jnp.pad(len_t, ((0, 0), (0, b_pad - b)), constant_values=1.0)

    tw = table.astype(jnp.float32) @ wenc.astype(jnp.float32)

    def _aug_t(w, bias, rows):
        m = jnp.concatenate(
            [w.astype(jnp.float32).T, bias.astype(jnp.float32).T], axis=1)
        r, c = m.shape
        return jnp.pad(m, ((0, rows - r), (0, 72 - c)))
    w_slab = jnp.concatenate([
        _aug_t(tw, benc, 64),
        _aug_t(w1, b1, 64),
        _aug_t(w2, b2, 64),
        _aug_t(w3, b3, 64),
    ], axis=0).astype(bf)

    out_t = pl.pallas_call(
        _encoder_kernel,
        out_shape=jax.ShapeDtypeStruct((_N_OUT, b_pad), jnp.float32),
        grid=(b_pad // cblk,),
        in_specs=[
            pl.BlockSpec((_S, cblk), lambda i: (0, i)),
            pl.BlockSpec((1, cblk), lambda i: (0, i)),
            pl.BlockSpec((256, 72), lambda i: (0, 0)),
        ],
        out_specs=pl.BlockSpec((_N_OUT, cblk), lambda i: (0, i)),
        compiler_params=pltpu.CompilerParams(
            dimension_semantics=("parallel",)),
    )(tok_t, len_t, w_slab)

    return {"pred_block_logits": out_t.T[:b]}


# speedup vs baseline: 1.0190x; 1.0190x over previous
import jax
import jax.numpy as jnp
from jax import lax
from jax.experimental import pallas as pl
from jax.experimental.pallas import tpu as pltpu

_S = 8
_V = 32
_N_OUT = 21
_CBLK = 32768


def _encoder_kernel(tok_ref, len_ref, w_ref, out_ref):
    f32, bf = jnp.float32, jnp.bfloat16
    cols = tok_ref.shape[1]

    lens = len_ref[0:1, :]
    tap_id = lax.broadcasted_iota(jnp.int32, (_S, 1), 0).astype(f32)
    toks = jnp.where(tap_id < lens, tok_ref[...].astype(bf),
                     jnp.array(-1.0, bf))

    vocab_id = lax.broadcasted_iota(jnp.int32, (_V, cols), 0).astype(bf)
    acc = jnp.zeros((_V, cols), bf)
    one, zero = jnp.array(1.0, bf), jnp.array(0.0, bf)
    for t in range(_S):
        hit = toks[t:t + 1, :] == vocab_id
        acc = acc + jnp.where(hit, one, zero)
    inv = (1.0 / jnp.maximum(lens, 1.0)).astype(bf)
    combined = acc * inv

    ones_row = jnp.full((1, cols), 1.0, bf)

    def aug(x):
        return jnp.concatenate([x, ones_row], axis=0)

    wenc_t = w_ref[0:_V, 0:_V + 1]
    w1_t = w_ref[64:128, 0:_V + 1]
    w2_t = w_ref[128:192, 0:65]
    w3_t = w_ref[192:192 + _N_OUT, 0:65]

    zero_bf = jnp.array(0.0, bf)
    sent = jnp.tanh(
        jnp.dot(wenc_t, aug(combined), preferred_element_type=f32))
    h1 = jnp.maximum(
        jnp.dot(w1_t, aug(sent.astype(bf)),
                preferred_element_type=f32).astype(bf), zero_bf)
    h2 = jnp.maximum(
        jnp.dot(w2_t, aug(h1), preferred_element_type=f32).astype(bf), zero_bf)
    out_ref[...] = jnp.dot(w3_t, aug(h2), preferred_element_type=f32)


@jax.jit
def kernel(tokens, lengths, table, wenc, benc, w1, b1, w2, b2, w3, b3):
    b, s = tokens.shape
    assert s == _S
    bf = jnp.bfloat16
    cblk = max(128, min(_CBLK, ((b + 127) // 128) * 128))
    b_pad = ((b + cblk - 1) // cblk) * cblk

    tok_t = tokens.astype(jnp.int32).T
    len_t = lengths.astype(jnp.float32).reshape(1, b)
    if b_pad != b:
        tok_t = jnp.pad(tok_t, ((0, 0), (0, b_pad - b)))
        len_t = jnp.pad(len_t, ((0, 0), (0, b_pad - b)), constant_values=1.0)

    tw = table.astype(jnp.float32) @ wenc.astype(jnp.float32)

    def _aug_t(w, bias, rows):
        m = jnp.concatenate(
            [w.astype(jnp.float32).T, bias.astype(jnp.float32).T], axis=1)
        r, c = m.shape
        return jnp.pad(m, ((0, rows - r), (0, 72 - c)))
    w_slab = jnp.concatenate([
        _aug_t(tw, benc, 64),
        _aug_t(w1, b1, 64),
        _aug_t(w2, b2, 64),
        _aug_t(w3, b3, 64),
    ], axis=0).astype(bf)

    out_t = pl.pallas_call(
        _encoder_kernel,
        out_shape=jax.ShapeDtypeStruct((_N_OUT, b_pad), jnp.float32),
        grid=(b_pad // cblk,),
        in_specs=[
            pl.BlockSpec((_S, cblk), lambda i: (0, i)),
            pl.BlockSpec((1, cblk), lambda i: (0, i)),
            pl.BlockSpec((256, 72), lambda i: (0, 0)),
        ],
        out_specs=pl.BlockSpec((_N_OUT, cblk), lambda i: (0, i)),
        compiler_params=pltpu.CompilerParams(
            dimension_semantics=("parallel",)),
    )(tok_t, len_t, w_slab)

    return {"pred_block_logits": out_t.T[:b]}
